# baseline (speedup 1.0000x reference)
import dataclasses

import jax
import jax.numpy as jnp
from jax import lax
from jax.experimental import pallas as pl
from jax.experimental.pallas import tpu as pltpu
from jax.experimental.pallas import tpu_sc as plsc

_N_CHAINS = 64
_N_SITES = 8192
_LANES = 16
_GRANULES_PER_CHAIN = _N_SITES // _LANES

_ROTS = (13, 15, 26, 6, 17, 29, 16, 24)


def _threefry2x32(k0, k1, x0, x1):
    ks = (k0, k1, k0 ^ k1 ^ jnp.int32(0x1BD11BDA))
    x0 = x0 + ks[0]
    x1 = x1 + ks[1]
    for g in range(5):
        for j in range(4):
            r = _ROTS[(g % 2) * 4 + j]
            x0 = x0 + x1
            x1 = (x1 << jnp.int32(r)) | lax.shift_right_logical(
                x1, jnp.int32(32 - r)
            )
            x1 = x0 ^ x1
        x0 = x0 + ks[(g + 1) % 3]
        x1 = x1 + ks[(g + 2) % 3] + jnp.int32(g + 1)
    return x0, x1


def _sc_body(x_hbm, seeds_hbm, out_hbm, seed_v, majd_v, gbuf, sem_copy, sem_a):
    q = lax.axis_index("c")
    s = lax.axis_index("s")
    w = q * 16 + s

    cp = pltpu.async_copy(
        x_hbm.at[pl.ds(w * 1024, 1024)],
        out_hbm.at[pl.ds(w * 1024, 1024)],
        sem_copy,
    )

    @pl.when(s < 2)
    def _rng_and_flip():
        grp = q * 2 + s
        pltpu.async_copy(
            seeds_hbm.at[pl.ds(grp * _LANES, _LANES)], seed_v, sem_a
        ).wait()
        sv = seed_v[...]

        zero = jnp.zeros((_LANES,), jnp.int32)
        one = zero + jnp.int32(1)
        f0, f1 = _threefry2x32(zero, zero, zero, sv)
        k20, k21 = _threefry2x32(f0, f1, zero, one)
        y0, y1 = _threefry2x32(k20, k21, zero, zero)
        idx = (y0 ^ y1) & jnp.int32(_N_SITES - 1)

        lanes = lax.iota(jnp.int32, _LANES)
        chain = grp * _LANES + lanes
        majd_v[...] = chain * _GRANULES_PER_CHAIN + lax.shift_right_logical(
            idx, jnp.int32(4)
        )
        lane = idx & jnp.int32(_LANES - 1)

        pltpu.async_copy(x_hbm.at[majd_v], gbuf, sem_a).wait()
        vals = plsc.load_gather(gbuf, [lanes, lane])
        plsc.store_scatter(gbuf, [lanes, lane], -vals)

    cp.wait()
    plsc.subcore_barrier()

    @pl.when(s < 2)
    def _write_flips():
        pltpu.async_copy(gbuf, out_hbm.at[majd_v], sem_a).wait()


_compiler_params = pltpu.CompilerParams(
    needs_layout_passes=False, use_tc_tiling_on_sc=False
)

_sc_call = pl.kernel(
    _sc_body,
    compiler_params=_compiler_params,
    out_type=jax.ShapeDtypeStruct(
        (_N_CHAINS * _GRANULES_PER_CHAIN, _LANES), jnp.float32
    ),
    mesh=plsc.VectorSubcoreMesh(
        core_axis_name="c", subcore_axis_name="s", num_cores=2, num_subcores=16
    ),
    scratch_types=[
        pltpu.VMEM((_LANES,), jnp.int32),
        pltpu.VMEM((_LANES,), jnp.int32),
        pltpu.VMEM((_LANES, _LANES), jnp.float32),
        pltpu.SemaphoreType.DMA,
        pltpu.SemaphoreType.DMA,
    ],
)


def kernel(x, seeds):
    x16 = x.reshape(_N_CHAINS * _GRANULES_PER_CHAIN, _LANES)
    out = _sc_call(x16, seeds)
    return out.reshape(_N_CHAINS, _N_SITES)


# speedup vs baseline: 3.4698x; 3.4698x over previous
import dataclasses

import jax
import jax.numpy as jnp
from jax import lax
from jax.experimental import pallas as pl
from jax.experimental.pallas import tpu as pltpu
from jax.experimental.pallas import tpu_sc as plsc

_N_CHAINS = 64
_N_SITES = 8192
_LANES = 16
_GRANULES_PER_CHAIN = _N_SITES // _LANES

_ROTS = (13, 15, 26, 6, 17, 29, 16, 24)


def _threefry2x32(k0, k1, x0, x1):
    ks = (k0, k1, k0 ^ k1 ^ jnp.int32(0x1BD11BDA))
    x0 = x0 + ks[0]
    x1 = x1 + ks[1]
    for g in range(5):
        for j in range(4):
            r = _ROTS[(g % 2) * 4 + j]
            x0 = x0 + x1
            x1 = (x1 << jnp.int32(r)) | lax.shift_right_logical(
                x1, jnp.int32(32 - r)
            )
            x1 = x0 ^ x1
        x0 = x0 + ks[(g + 1) % 3]
        x1 = x1 + ks[(g + 2) % 3] + jnp.int32(g + 1)
    return x0, x1


def _sc_body(x_hbm, seeds_hbm, out_hbm, seed_v, row_buf, sem_in, sem_seed,
             sem_out):
    q = lax.axis_index("c")
    s = lax.axis_index("s")
    w = q * 16 + s
    base = w * 1024

    cin = pltpu.async_copy(x_hbm.at[pl.ds(base, 1024)], row_buf, sem_in)

    grp = w >> 3
    pltpu.async_copy(
        seeds_hbm.at[pl.ds(grp * _LANES, _LANES)], seed_v, sem_seed
    ).wait()
    sv = seed_v[...]

    zero = jnp.zeros((_LANES,), jnp.int32)
    one = zero + jnp.int32(1)
    f0, f1 = _threefry2x32(zero, zero, zero, sv)
    k20, k21 = _threefry2x32(f0, f1, zero, one)
    y0, y1 = _threefry2x32(k20, k21, zero, zero)
    idx = (y0 ^ y1) & jnp.int32(_N_SITES - 1)

    lanes = lax.iota(jnp.int32, _LANES)
    l0 = (w & 7) * 2
    mask = (lanes >= l0) & (lanes < l0 + 2)
    lgran = jnp.where(
        mask,
        (lanes - l0) * _GRANULES_PER_CHAIN
        + lax.shift_right_logical(idx, jnp.int32(4)),
        0,
    )
    lane = jnp.where(mask, idx & jnp.int32(_LANES - 1), 0)

    cin.wait()
    vals = plsc.load_gather(row_buf, [lgran, lane], mask=mask)
    plsc.store_scatter(row_buf, [lgran, lane], -vals, mask=mask)
    pltpu.async_copy(row_buf, out_hbm.at[pl.ds(base, 1024)], sem_out).wait()


_compiler_params = pltpu.CompilerParams(
    needs_layout_passes=False, use_tc_tiling_on_sc=False
)

_sc_call = pl.kernel(
    _sc_body,
    compiler_params=_compiler_params,
    out_type=jax.ShapeDtypeStruct(
        (_N_CHAINS * _GRANULES_PER_CHAIN, _LANES), jnp.float32
    ),
    mesh=plsc.VectorSubcoreMesh(
        core_axis_name="c", subcore_axis_name="s", num_cores=2, num_subcores=16
    ),
    scratch_types=[
        pltpu.VMEM((_LANES,), jnp.int32),
        pltpu.VMEM((1024, _LANES), jnp.float32),
        pltpu.SemaphoreType.DMA,
        pltpu.SemaphoreType.DMA,
        pltpu.SemaphoreType.DMA,
    ],
)


def kernel(x, seeds):
    x16 = x.reshape(_N_CHAINS * _GRANULES_PER_CHAIN, _LANES)
    out = _sc_call(x16, seeds)
    return out.reshape(_N_CHAINS, _N_SITES)
